# baseline (speedup 1.0000x reference)
import functools

import jax
import jax.numpy as jnp
from jax import lax
from jax.experimental import pallas as pl
from jax.experimental.pallas import tpu as pltpu
from jax.experimental.pallas import tpu_sc as plsc

NC = 2
NS = 16
NW = NC * NS
CB = 128


def _fill_2d(ref, rows, cols, value):
    per_row = cols // 16

    def body(k, carry):
        ref[k // per_row, pl.ds((k % per_row) * 16, 16)] = jnp.full(
            (16,), value, jnp.float32)
        return carry

    lax.fori_loop(0, rows * per_row, body, 0)


def _degree_partials(row3, npad, d):
    nch = row3.shape[1]
    sl = npad // NS

    mesh = plsc.VectorSubcoreMesh(core_axis_name="c", subcore_axis_name="s")

    @functools.partial(
        pl.kernel,
        out_type=jax.ShapeDtypeStruct((NC, npad, d), jnp.float32),
        mesh=mesh,
        scratch_types=[
            pltpu.VMEM((nch, CB), jnp.int32),
            pltpu.VMEM((CB, d), jnp.float32),
            pltpu.SemaphoreType.DMA,
            pltpu.VMEM_SHARED((npad, d), jnp.float32),
        ],
    )
    def hist(row_hbm, deg_out, rows_v, buf, sem, deg_sh):
        c = lax.axis_index("c")
        s = lax.axis_index("s")
        wid = c * NS + s
        _fill_2d(buf, CB, d, 0.0)

        def zrow(k, carry):
            pltpu.sync_copy(buf, deg_sh.at[pl.ds(s * sl + k * CB, CB)])
            return carry

        lax.fori_loop(0, sl // CB, zrow, 0)
        _fill_2d(buf, CB, d, 1.0)
        plsc.subcore_barrier()
        pltpu.sync_copy(row_hbm.at[wid], rows_v)

        k_fly = 4
        for j in range(k_fly):
            pltpu.async_copy(buf, deg_sh.at[rows_v.at[j]], sem, add=True)

        def chunk(j, carry):
            pltpu.make_async_copy(buf, deg_sh.at[rows_v.at[0]], sem).wait()
            pltpu.async_copy(buf, deg_sh.at[rows_v.at[j]], sem, add=True)
            return carry

        lax.fori_loop(k_fly, nch, chunk, 0)
        for _ in range(k_fly):
            pltpu.make_async_copy(buf, deg_sh.at[rows_v.at[0]], sem).wait()
        plsc.subcore_barrier()
        pltpu.sync_copy(deg_sh.at[pl.ds(s * sl, sl)],
                        deg_out.at[c, pl.ds(s * sl, sl)])

    return hist(row3)


def _scale_x(degp, x_pad):
    npad, d = x_pad.shape
    br = 1024

    def body(deg_ref, x_ref, xs_ref):
        deg = deg_ref[0, :, 0:1] + deg_ref[1, :, 0:1]
        xs_ref[...] = x_ref[...] * lax.rsqrt(deg)

    return pl.pallas_call(
        body,
        grid=(npad // br,),
        in_specs=[
            pl.BlockSpec((NC, br, d), lambda i: (0, i, 0)),
            pl.BlockSpec((br, d), lambda i: (i, 0)),
        ],
        out_specs=pl.BlockSpec((br, d), lambda i: (i, 0)),
        out_shape=jax.ShapeDtypeStruct((npad, d), jnp.float32),
    )(degp, x_pad)


def _aggregate_partials(xs, col3, row3):
    nch = col3.shape[1]
    npad, d = xs.shape
    sl = npad // NS

    mesh = plsc.VectorSubcoreMesh(core_axis_name="c", subcore_axis_name="s")

    @functools.partial(
        pl.kernel,
        out_type=jax.ShapeDtypeStruct((NC, npad, d), jnp.float32),
        mesh=mesh,
        scratch_types=[
            pltpu.VMEM((nch, CB), jnp.int32),
            pltpu.VMEM((2, CB), jnp.int32),
            pltpu.VMEM((CB, d), jnp.float32),
            pltpu.VMEM((CB, d), jnp.float32),
            pltpu.SemaphoreType.DMA,
            pltpu.SemaphoreType.DMA,
            pltpu.SemaphoreType.DMA,
            pltpu.SemaphoreType.DMA,
            pltpu.SemaphoreType.DMA,
            pltpu.SemaphoreType.DMA,
            pltpu.VMEM_SHARED((npad, d), jnp.float32),
        ],
    )
    def agg(xs_hbm, col_hbm, row_hbm, agg_out, cols_v, rows2, buf0, buf1,
            gsem0, gsem1, rsem0, rsem1, ssem0, ssem1, agg_sh):
        c = lax.axis_index("c")
        s = lax.axis_index("s")
        wid = c * NS + s
        bufs = (buf0, buf1)
        gsems = (gsem0, gsem1)
        rsems = (rsem0, rsem1)
        ssems = (ssem0, ssem1)
        _fill_2d(buf0, CB, d, 0.0)

        def zrow(k, carry):
            pltpu.sync_copy(buf0, agg_sh.at[pl.ds(s * sl + k * CB, CB)])
            return carry

        lax.fori_loop(0, sl // CB, zrow, 0)
        plsc.subcore_barrier()
        pltpu.sync_copy(col_hbm.at[wid], cols_v)

        pltpu.async_copy(xs_hbm.at[cols_v.at[0]], buf0, gsem0)
        pltpu.async_copy(row_hbm.at[wid, 0], rows2.at[0], rsem0)

        def pair(g, carry):
            for b in range(2):
                j = 2 * g + b
                pltpu.make_async_copy(
                    xs_hbm.at[cols_v.at[j]], bufs[b], gsems[b]).wait()
                pltpu.make_async_copy(
                    row_hbm.at[wid, j], rows2.at[b], rsems[b]).wait()
                pltpu.async_copy(
                    bufs[b], agg_sh.at[rows2.at[b]], ssems[b], add=True)

                @pl.when(j + 1 < nch)
                def _():
                    @pl.when(j >= 1)
                    def _():
                        pltpu.make_async_copy(
                            bufs[1 - b], agg_sh.at[rows2.at[1 - b]],
                            ssems[1 - b]).wait()

                    pltpu.async_copy(
                        xs_hbm.at[cols_v.at[j + 1]], bufs[1 - b],
                        gsems[1 - b])
                    pltpu.async_copy(
                        row_hbm.at[wid, j + 1], rows2.at[1 - b],
                        rsems[1 - b])
            return carry

        lax.fori_loop(0, nch // 2, pair, 0)
        pltpu.make_async_copy(
            bufs[0], agg_sh.at[rows2.at[0]], ssems[0]).wait()
        pltpu.make_async_copy(
            bufs[1], agg_sh.at[rows2.at[1]], ssems[1]).wait()
        plsc.subcore_barrier()
        pltpu.sync_copy(agg_sh.at[pl.ds(s * sl, sl)],
                        agg_out.at[c, pl.ds(s * sl, sl)])

    return agg(xs, col3, row3)


def _combine(degp, aggp):
    _, npad, d = aggp.shape
    br = 1024

    def body(deg_ref, agg_ref, out_ref):
        deg = deg_ref[0, :, 0:1] + deg_ref[1, :, 0:1]
        dinv = lax.rsqrt(deg)
        out_ref[...] = (agg_ref[0] + agg_ref[1]) * (dinv * dinv * dinv)

    return pl.pallas_call(
        body,
        grid=(npad // br,),
        in_specs=[
            pl.BlockSpec((NC, br, d), lambda i: (0, i, 0)),
            pl.BlockSpec((NC, br, d), lambda i: (0, i, 0)),
        ],
        out_specs=pl.BlockSpec((br, d), lambda i: (i, 0)),
        out_shape=jax.ShapeDtypeStruct((npad, d), jnp.float32),
    )(degp, aggp)


def kernel(x, edge_index):
    n, d = x.shape
    e = edge_index.shape[1]
    per_tile = -(-e // NW // (2 * CB)) * (2 * CB)
    nch = per_tile // CB
    align = NS * CB
    npad = ((n + align - 1) // align) * align
    if npad == n:
        npad += align

    e_pad = per_tile * NW
    rows = edge_index[0]
    cols = edge_index[1]
    if e_pad != e:
        pad_iota = jnp.arange(e_pad - e, dtype=jnp.int32)
        trash = jnp.int32(n) + pad_iota % jnp.int32(npad - n)
        rows = jnp.concatenate([rows, trash])
        cols = jnp.concatenate([cols, pad_iota % jnp.int32(n)])
    row3 = rows.reshape(NW, nch, CB)
    col3 = cols.reshape(NW, nch, CB)
    x_pad = jnp.zeros((npad, d), jnp.float32).at[:n].set(x)

    degp = _degree_partials(row3, npad, d)
    xs = _scale_x(degp, x_pad)
    aggp = _aggregate_partials(xs, col3, row3)
    out = _combine(degp, aggp)
    return out[:n]


# speedup vs baseline: 1.0980x; 1.0980x over previous
import functools

import jax
import jax.numpy as jnp
from jax import lax
from jax.experimental import pallas as pl
from jax.experimental.pallas import tpu as pltpu
from jax.experimental.pallas import tpu_sc as plsc

NC = 2
NS = 16
NW = NC * NS
CB = 128


def _fill_2d(ref, rows, cols, value):
    per_row = cols // 16

    def body(k, carry):
        ref[k // per_row, pl.ds((k % per_row) * 16, 16)] = jnp.full(
            (16,), value, jnp.float32)
        return carry

    lax.fori_loop(0, rows * per_row, body, 0)


def _degree_partials(row3, npad, d):
    nch = row3.shape[1]
    sl = npad // NS

    mesh = plsc.VectorSubcoreMesh(core_axis_name="c", subcore_axis_name="s")

    @functools.partial(
        pl.kernel,
        out_type=jax.ShapeDtypeStruct((NC, npad, d), jnp.float32),
        mesh=mesh,
        scratch_types=[
            pltpu.VMEM((nch, CB), jnp.int32),
            pltpu.VMEM((CB, d), jnp.float32),
            pltpu.SemaphoreType.DMA,
            pltpu.VMEM_SHARED((npad, d), jnp.float32),
        ],
    )
    def hist(row_hbm, deg_out, rows_v, buf, sem, deg_sh):
        c = lax.axis_index("c")
        s = lax.axis_index("s")
        wid = c * NS + s
        _fill_2d(buf, CB, d, 0.0)

        def zrow(k, carry):
            pltpu.sync_copy(buf, deg_sh.at[pl.ds(s * sl + k * CB, CB)])
            return carry

        lax.fori_loop(0, sl // CB, zrow, 0)
        _fill_2d(buf, CB, d, 1.0)
        plsc.subcore_barrier()
        pltpu.sync_copy(row_hbm.at[wid], rows_v)

        k_fly = 4
        for j in range(k_fly):
            pltpu.async_copy(buf, deg_sh.at[rows_v.at[j]], sem, add=True)

        def chunk(j, carry):
            pltpu.make_async_copy(buf, deg_sh.at[rows_v.at[0]], sem).wait()
            pltpu.async_copy(buf, deg_sh.at[rows_v.at[j]], sem, add=True)
            return carry

        lax.fori_loop(k_fly, nch, chunk, 0)
        for _ in range(k_fly):
            pltpu.make_async_copy(buf, deg_sh.at[rows_v.at[0]], sem).wait()
        plsc.subcore_barrier()
        pltpu.sync_copy(deg_sh.at[pl.ds(s * sl, sl)],
                        deg_out.at[c, pl.ds(s * sl, sl)])

    return hist(row3)


def _scale_x(degp, x_pad):
    npad, d = x_pad.shape
    br = 1024

    def body(deg_ref, x_ref, xs_ref):
        deg = deg_ref[0, :, 0:1] + deg_ref[1, :, 0:1]
        xs_ref[...] = x_ref[...] * lax.rsqrt(deg)

    return pl.pallas_call(
        body,
        grid=(npad // br,),
        in_specs=[
            pl.BlockSpec((NC, br, d), lambda i: (0, i, 0)),
            pl.BlockSpec((br, d), lambda i: (i, 0)),
        ],
        out_specs=pl.BlockSpec((br, d), lambda i: (i, 0)),
        out_shape=jax.ShapeDtypeStruct((npad, d), jnp.float32),
    )(degp, x_pad)


def _aggregate_partials(xs, col3, row3):
    nch = col3.shape[1]
    npad, d = xs.shape
    sl = npad // NS

    mesh = plsc.VectorSubcoreMesh(core_axis_name="c", subcore_axis_name="s")

    @functools.partial(
        pl.kernel,
        out_type=jax.ShapeDtypeStruct((NC, npad, d), jnp.float32),
        mesh=mesh,
        scratch_types=[
            pltpu.VMEM((nch, CB), jnp.int32),
            pltpu.VMEM((2, CB), jnp.int32),
            pltpu.VMEM((CB, d), jnp.float32),
            pltpu.VMEM((CB, d), jnp.float32),
            pltpu.SemaphoreType.DMA,
            pltpu.SemaphoreType.DMA,
            pltpu.SemaphoreType.DMA,
            pltpu.SemaphoreType.DMA,
            pltpu.VMEM_SHARED((npad, d), jnp.float32),
        ],
    )
    def agg(xs_hbm, col_hbm, row_hbm, agg_out, cols_v, rows2, buf0, buf1,
            gsem0, gsem1, rsem0, rsem1, agg_sh):
        c = lax.axis_index("c")
        s = lax.axis_index("s")
        wid = c * NS + s
        bufs = (buf0, buf1)
        gsems = (gsem0, gsem1)
        rsems = (rsem0, rsem1)
        _fill_2d(buf0, CB, d, 0.0)

        def zrow(k, carry):
            pltpu.sync_copy(buf0, agg_sh.at[pl.ds(s * sl + k * CB, CB)])
            return carry

        lax.fori_loop(0, sl // CB, zrow, 0)
        plsc.subcore_barrier()
        pltpu.sync_copy(col_hbm.at[wid], cols_v)

        pltpu.async_copy(xs_hbm.at[cols_v.at[0]], buf0, gsem0)
        pltpu.async_copy(row_hbm.at[wid, 0], rows2.at[0], rsem0)

        def pair(g, carry):
            for b in range(2):
                j = 2 * g + b

                @pl.when(j + 1 < nch)
                def _():
                    pltpu.async_copy(
                        xs_hbm.at[cols_v.at[j + 1]], bufs[1 - b],
                        gsems[1 - b])
                    pltpu.async_copy(
                        row_hbm.at[wid, j + 1], rows2.at[1 - b],
                        rsems[1 - b])

                pltpu.make_async_copy(
                    xs_hbm.at[cols_v.at[j]], bufs[b], gsems[b]).wait()
                pltpu.make_async_copy(
                    row_hbm.at[wid, j], rows2.at[b], rsems[b]).wait()
                pltpu.sync_copy(bufs[b], agg_sh.at[rows2.at[b]], add=True)
            return carry

        lax.fori_loop(0, nch // 2, pair, 0)
        plsc.subcore_barrier()
        pltpu.sync_copy(agg_sh.at[pl.ds(s * sl, sl)],
                        agg_out.at[c, pl.ds(s * sl, sl)])

    return agg(xs, col3, row3)


def _combine(degp, aggp):
    _, npad, d = aggp.shape
    br = 1024

    def body(deg_ref, agg_ref, out_ref):
        deg = deg_ref[0, :, 0:1] + deg_ref[1, :, 0:1]
        dinv = lax.rsqrt(deg)
        out_ref[...] = (agg_ref[0] + agg_ref[1]) * (dinv * dinv * dinv)

    return pl.pallas_call(
        body,
        grid=(npad // br,),
        in_specs=[
            pl.BlockSpec((NC, br, d), lambda i: (0, i, 0)),
            pl.BlockSpec((NC, br, d), lambda i: (0, i, 0)),
        ],
        out_specs=pl.BlockSpec((br, d), lambda i: (i, 0)),
        out_shape=jax.ShapeDtypeStruct((npad, d), jnp.float32),
    )(degp, aggp)


def kernel(x, edge_index):
    n, d = x.shape
    e = edge_index.shape[1]
    per_tile = -(-e // NW // (2 * CB)) * (2 * CB)
    nch = per_tile // CB
    align = NS * CB
    npad = ((n + align - 1) // align) * align
    if npad == n:
        npad += align

    e_pad = per_tile * NW
    rows = edge_index[0]
    cols = edge_index[1]
    if e_pad != e:
        pad_iota = jnp.arange(e_pad - e, dtype=jnp.int32)
        trash = jnp.int32(n) + pad_iota % jnp.int32(npad - n)
        rows = jnp.concatenate([rows, trash])
        cols = jnp.concatenate([cols, pad_iota % jnp.int32(n)])
    row3 = rows.reshape(NW, nch, CB)
    col3 = cols.reshape(NW, nch, CB)
    x_pad = jnp.zeros((npad, d), jnp.float32).at[:n].set(x)

    degp = _degree_partials(row3, npad, d)
    xs = _scale_x(degp, x_pad)
    aggp = _aggregate_partials(xs, col3, row3)
    out = _combine(degp, aggp)
    return out[:n]
